# baseline (speedup 1.0000x reference)
import functools

import jax
import jax.numpy as jnp
from jax import lax
from jax.experimental import pallas as pl
from jax.experimental.pallas import tpu as pltpu
from jax.experimental.pallas import tpu_sc as plsc

N = 10000
E = 320000
NC, NS = 2, 16
NW = NC * NS
EPT = E // NW
K128 = 50
NB128 = 4
K16 = 50
NB16 = 10
RPT = 624
RTAIL = N - NS * RPT

_MESH = plsc.VectorSubcoreMesh(
    core_axis_name="c", subcore_axis_name="s", num_cores=NC, num_subcores=NS
)


def _sc_edge_aggregate(D, with_gather, K, NB):
    NCH = EPT // K
    NOUTER = NCH // NB
    assert NCH * K == EPT and NOUTER * NB == NCH
    ZCOPIES = RPT // K
    ZREM = RPT % K
    scratch = [
        pltpu.VMEM((NCH, K), jnp.int32),
        pltpu.VMEM((NCH, K), jnp.int32),
        [pltpu.VMEM((K, D), jnp.float32) for _ in range(NB)],
        pltpu.VMEM_SHARED((N, D), jnp.float32),
        [pltpu.SemaphoreType.DMA for _ in range(NB)],
        [pltpu.SemaphoreType.DMA for _ in range(NB)],
    ]
    if D == 128:
        out_type = [
            jax.ShapeDtypeStruct((N, 128), jnp.float32),
            jax.ShapeDtypeStruct((N, 128), jnp.float32),
        ]
    else:
        out_type = [jax.ShapeDtypeStruct((N, 128), jnp.float32)]

    @functools.partial(
        pl.kernel, out_type=out_type, mesh=_MESH, scratch_types=scratch,
        compiler_params=pltpu.CompilerParams(use_tc_tiling_on_sc=False),
    )
    def k(edge_hbm, table_hbm, *rest):
        if D == 128:
            out0, out1 = rest[0], rest[1]
            src_all, dst_all, rows, acc_sh, sem_g, sem_s = rest[2:]
        else:
            out0 = rest[0]
            src_all, dst_all, rows, acc_sh, sem_g, sem_s = rest[1:]
        c = lax.axis_index("c")
        s = lax.axis_index("s")
        wid = s * NC + c

        zeros = jnp.zeros((16,), jnp.float32)

        def zrow(i, carry):
            for j in range(D // 16):
                rows[0][i, pl.ds(j * 16, 16)] = zeros
            return carry

        lax.fori_loop(0, K, zrow, 0)

        def zinit(j, n):
            return pltpu.make_async_copy(
                rows[0].at[pl.ds(0, n)],
                acc_sh.at[pl.ds(s * RPT + j * K, n)], sem_s[j % NB])

        for j in range(ZCOPIES):
            zinit(j, K).start()
        pltpu.sync_copy(rows[0].at[pl.ds(0, ZREM)],
                        acc_sh.at[pl.ds(s * RPT + ZCOPIES * K, ZREM)])

        @pl.when(s == NS - 1)
        def _():
            pltpu.sync_copy(rows[0].at[pl.ds(0, RTAIL)],
                            acc_sh.at[pl.ds(NS * RPT, RTAIL)])

        pltpu.sync_copy(edge_hbm.at[1, wid], dst_all)
        if with_gather:
            pltpu.sync_copy(edge_hbm.at[0, wid], src_all)
        for j in range(ZCOPIES):
            zinit(j, K).wait()

        if not with_gather:
            ones = jnp.ones((16,), jnp.float32)

            def orow(i, carry):
                for b in range(NB):
                    for j in range(D // 16):
                        rows[b][i, pl.ds(j * 16, 16)] = ones
                return carry

            lax.fori_loop(0, K, orow, 0)

        plsc.subcore_barrier()

        def gather_start(b, g):
            pltpu.async_copy(table_hbm.at[src_all.at[g]], rows[b], sem_g[b])

        def gather_wait(b, g):
            pltpu.make_async_copy(
                table_hbm.at[src_all.at[g]], rows[b], sem_g[b]).wait()

        def scatter_start(b, g):
            pltpu.async_copy(rows[b], acc_sh.at[dst_all.at[g]], sem_s[b],
                             add=True)

        def scatter_wait(b, g):
            pltpu.make_async_copy(
                rows[b], acc_sh.at[dst_all.at[g]], sem_s[b]).wait()

        if with_gather:
            for b in range(NB):
                gather_start(b, b)

            def body(t, carry):
                for b in range(NB):
                    g = t * NB + b
                    gather_wait(b, g)
                    scatter_start(b, g)

                    @pl.when(g + NB < NCH)
                    def _():
                        scatter_wait(b, g)
                        gather_start(b, g + NB)
                return carry

            lax.fori_loop(0, NOUTER, body, 0)
            for b in range(NB):
                scatter_wait(b, NCH - NB + b)
        else:
            def body(t, carry):
                for b in range(NB):
                    g = t * NB + b

                    @pl.when(t > 0)
                    def _():
                        scatter_wait(b, g)
                    scatter_start(b, g)
                return carry

            lax.fori_loop(0, NOUTER, body, 0)
            for b in range(NB):
                scatter_wait(b, NCH - NB + b)

        plsc.subcore_barrier()

        if D == 128:
            @pl.when(c == 0)
            def _():
                pltpu.sync_copy(acc_sh.at[pl.ds(s * RPT, RPT)],
                                out0.at[pl.ds(s * RPT, RPT)])

                @pl.when(s == NS - 1)
                def _():
                    pltpu.sync_copy(acc_sh.at[pl.ds(NS * RPT, RTAIL)],
                                    out0.at[pl.ds(NS * RPT, RTAIL)])

            @pl.when(c == 1)
            def _():
                pltpu.sync_copy(acc_sh.at[pl.ds(s * RPT, RPT)],
                                out1.at[pl.ds(s * RPT, RPT)])

                @pl.when(s == NS - 1)
                def _():
                    pltpu.sync_copy(acc_sh.at[pl.ds(NS * RPT, RTAIL)],
                                    out1.at[pl.ds(NS * RPT, RTAIL)])
        else:
            @pl.when(c == 0)
            def _():
                pltpu.sync_copy(acc_sh.at[pl.ds(s * RPT, RPT)],
                                out0.at[pl.ds(s * RPT, RPT), pl.ds(0, D)])

                @pl.when(s == NS - 1)
                def _():
                    pltpu.sync_copy(
                        acc_sh.at[pl.ds(NS * RPT, RTAIL)],
                        out0.at[pl.ds(NS * RPT, RTAIL), pl.ds(0, D)])

            @pl.when(c == 1)
            def _():
                pltpu.sync_copy(acc_sh.at[pl.ds(s * RPT, RPT)],
                                out0.at[pl.ds(s * RPT, RPT), pl.ds(D, D)])

                @pl.when(s == NS - 1)
                def _():
                    pltpu.sync_copy(
                        acc_sh.at[pl.ds(NS * RPT, RTAIL)],
                        out0.at[pl.ds(NS * RPT, RTAIL), pl.ds(D, D)])

    return k


_agg128 = _sc_edge_aggregate(128, True, K128, NB128)
_agg16 = _sc_edge_aggregate(16, True, K16, NB16)
_hist16 = _sc_edge_aggregate(16, False, K16, NB16)


_RB = 2000


def _tc_stage1(cnt, x, W1):
    def body(c_ref, x_ref, w_ref, hs_ref, dinv_ref):
        deg = c_ref[:, 0:1] + c_ref[:, 16:17] + 1.0
        dinv = lax.rsqrt(deg)
        h = jnp.dot(x_ref[...], w_ref[...], preferred_element_type=jnp.float32)
        hs_ref[...] = h * dinv
        dinv_ref[...] = dinv

    grid = (N // _RB,)
    return pl.pallas_call(
        body,
        grid=grid,
        in_specs=[
            pl.BlockSpec((_RB, 128), lambda i: (i, 0)),
            pl.BlockSpec((_RB, 128), lambda i: (i, 0)),
            pl.BlockSpec((128, 128), lambda i: (0, 0)),
        ],
        out_specs=[
            pl.BlockSpec((_RB, 128), lambda i: (i, 0)),
            pl.BlockSpec((_RB, 1), lambda i: (i, 0)),
        ],
        out_shape=[
            jax.ShapeDtypeStruct((N, 128), jnp.float32),
            jax.ShapeDtypeStruct((N, 1), jnp.float32),
        ],
    )(cnt, x, W1)


def _tc_stage2(p0, p1, hs1, dinv, b1, W2, Wc):
    def body(p0_ref, p1_ref, hs_ref, dinv_ref, b1_ref, w2_ref, wc_ref, out_ref):
        dinv = dinv_ref[...]
        y = (p0_ref[...] + p1_ref[...] + hs_ref[...]) * dinv + b1_ref[...]
        y = jnp.maximum(y, 0.0)
        h2 = jnp.dot(
            jnp.dot(y, w2_ref[...], preferred_element_type=jnp.float32),
            wc_ref[...], preferred_element_type=jnp.float32)
        hs2 = h2 * dinv
        out_ref[...] = jnp.pad(hs2, ((0, 0), (0, 14)))

    grid = (N // _RB,)
    return pl.pallas_call(
        body,
        grid=grid,
        in_specs=[
            pl.BlockSpec((_RB, 128), lambda i: (i, 0)),
            pl.BlockSpec((_RB, 128), lambda i: (i, 0)),
            pl.BlockSpec((_RB, 128), lambda i: (i, 0)),
            pl.BlockSpec((_RB, 1), lambda i: (i, 0)),
            pl.BlockSpec((1, 128), lambda i: (0, 0)),
            pl.BlockSpec((128, 64), lambda i: (0, 0)),
            pl.BlockSpec((64, 2), lambda i: (0, 0)),
        ],
        out_specs=pl.BlockSpec((_RB, 16), lambda i: (i, 0)),
        out_shape=jax.ShapeDtypeStruct((N, 16), jnp.float32),
    )(p0, p1, hs1, dinv, b1, W2, Wc)


def _tc_stage3(q, hs2p, dinv, b2, Wc, bc):
    def body(q_ref, hs_ref, dinv_ref, b2_ref, wc_ref, bc_ref, out_ref):
        agg = ((q_ref[:, 0:8] + q_ref[:, 16:24] + hs_ref[:, 0:8])
               * dinv_ref[...])
        b2c = jnp.dot(b2_ref[...], wc_ref[...],
                      preferred_element_type=jnp.float32) + bc_ref[...]
        out_ref[...] = agg[:, 0:2] + b2c

    grid = (N // _RB,)
    return pl.pallas_call(
        body,
        grid=grid,
        in_specs=[
            pl.BlockSpec((_RB, 128), lambda i: (i, 0)),
            pl.BlockSpec((_RB, 16), lambda i: (i, 0)),
            pl.BlockSpec((_RB, 1), lambda i: (i, 0)),
            pl.BlockSpec((1, 64), lambda i: (0, 0)),
            pl.BlockSpec((64, 2), lambda i: (0, 0)),
            pl.BlockSpec((1, 2), lambda i: (0, 0)),
        ],
        out_specs=pl.BlockSpec((_RB, 2), lambda i: (i, 0)),
        out_shape=jax.ShapeDtypeStruct((N, 2), jnp.float32),
    )(q, hs2p, dinv, b2, Wc, bc)


def _unwrap(res):
    return res[0] if isinstance(res, (list, tuple)) else res


def kernel(x, edge_index, W1, b1, W2, b2, Wc, bc):
    ei = edge_index.astype(jnp.int32)
    edge1 = ei.reshape(2, NW, EPT // K128, K128)
    edge2 = ei.reshape(2, NW, EPT // K16, K16)

    cnt = _unwrap(_hist16(edge2, x))

    hs1, dinv = _tc_stage1(cnt, x, W1)
    p0, p1 = _agg128(edge1, hs1)
    hs2p = _tc_stage2(p0, p1, hs1, dinv, b1.reshape(1, 128), W2, Wc)
    q = _unwrap(_agg16(edge2, hs2p))
    out = _tc_stage3(q, hs2p, dinv, b2.reshape(1, 64), Wc,
                     bc.reshape(1, 2))
    return out


# speedup vs baseline: 1.1083x; 1.1083x over previous
import functools

import jax
import jax.numpy as jnp
from jax import lax
from jax.experimental import pallas as pl
from jax.experimental.pallas import tpu as pltpu
from jax.experimental.pallas import tpu_sc as plsc

N = 10000
E = 320000
NC, NS = 2, 16
NW = NC * NS
EPT = E // NW
K128 = 40
NB128 = 5
K16 = 80
NB16 = 5
RPT = 624
RTAIL = N - NS * RPT

_MESH = plsc.VectorSubcoreMesh(
    core_axis_name="c", subcore_axis_name="s", num_cores=NC, num_subcores=NS
)


def _sc_edge_aggregate(D, with_gather, K, NB):
    NCH = EPT // K
    NOUTER = NCH // NB
    assert NCH * K == EPT and NOUTER * NB == NCH
    ZCOPIES = RPT // K
    ZREM = RPT % K
    scratch = [
        pltpu.VMEM((NCH, K), jnp.int32),
        pltpu.VMEM((NCH, K), jnp.int32),
        [pltpu.VMEM((K, D), jnp.float32) for _ in range(NB)],
        pltpu.VMEM_SHARED((N, D), jnp.float32),
        [pltpu.SemaphoreType.DMA for _ in range(NB)],
        [pltpu.SemaphoreType.DMA for _ in range(NB)],
    ]
    if D == 128:
        out_type = [
            jax.ShapeDtypeStruct((N, 128), jnp.float32),
            jax.ShapeDtypeStruct((N, 128), jnp.float32),
        ]
    else:
        out_type = [jax.ShapeDtypeStruct((N, 128), jnp.float32)]

    @functools.partial(
        pl.kernel, out_type=out_type, mesh=_MESH, scratch_types=scratch,
        compiler_params=pltpu.CompilerParams(use_tc_tiling_on_sc=False),
    )
    def k(edge_hbm, table_hbm, *rest):
        if D == 128:
            out0, out1 = rest[0], rest[1]
            src_all, dst_all, rows, acc_sh, sem_g, sem_s = rest[2:]
        else:
            out0 = rest[0]
            src_all, dst_all, rows, acc_sh, sem_g, sem_s = rest[1:]
        c = lax.axis_index("c")
        s = lax.axis_index("s")
        wid = s * NC + c

        zeros = jnp.zeros((16,), jnp.float32)

        def zrow(i, carry):
            for j in range(D // 16):
                rows[0][i, pl.ds(j * 16, 16)] = zeros
            return carry

        lax.fori_loop(0, K, zrow, 0)

        def zinit(j, n):
            return pltpu.make_async_copy(
                rows[0].at[pl.ds(0, n)],
                acc_sh.at[pl.ds(s * RPT + j * K, n)], sem_s[j % NB])

        for j in range(ZCOPIES):
            zinit(j, K).start()
        pltpu.sync_copy(rows[0].at[pl.ds(0, ZREM)],
                        acc_sh.at[pl.ds(s * RPT + ZCOPIES * K, ZREM)])

        @pl.when(s == NS - 1)
        def _():
            pltpu.sync_copy(rows[0].at[pl.ds(0, RTAIL)],
                            acc_sh.at[pl.ds(NS * RPT, RTAIL)])

        pltpu.sync_copy(edge_hbm.at[1, wid], dst_all)
        if with_gather:
            pltpu.sync_copy(edge_hbm.at[0, wid], src_all)
        for j in range(ZCOPIES):
            zinit(j, K).wait()

        if not with_gather:
            ones = jnp.ones((16,), jnp.float32)

            def orow(i, carry):
                for b in range(NB):
                    for j in range(D // 16):
                        rows[b][i, pl.ds(j * 16, 16)] = ones
                return carry

            lax.fori_loop(0, K, orow, 0)

        plsc.subcore_barrier()

        def gather_start(b, g):
            pltpu.async_copy(table_hbm.at[src_all.at[g]], rows[b], sem_g[b])

        def gather_wait(b, g):
            pltpu.make_async_copy(
                table_hbm.at[src_all.at[g]], rows[b], sem_g[b]).wait()

        def scatter_start(b, g):
            pltpu.async_copy(rows[b], acc_sh.at[dst_all.at[g]], sem_s[b],
                             add=True)

        def scatter_wait(b, g):
            pltpu.make_async_copy(
                rows[b], acc_sh.at[dst_all.at[g]], sem_s[b]).wait()

        if with_gather:
            for b in range(NB):
                gather_start(b, b)

            def body(t, carry):
                for b in range(NB):
                    g = t * NB + b
                    gather_wait(b, g)
                    scatter_start(b, g)

                    @pl.when(g + NB < NCH)
                    def _():
                        scatter_wait(b, g)
                        gather_start(b, g + NB)
                return carry

            lax.fori_loop(0, NOUTER, body, 0)
            for b in range(NB):
                scatter_wait(b, NCH - NB + b)
        else:
            def body(t, carry):
                for b in range(NB):
                    g = t * NB + b

                    @pl.when(t > 0)
                    def _():
                        scatter_wait(b, g)
                    scatter_start(b, g)
                return carry

            lax.fori_loop(0, NOUTER, body, 0)
            for b in range(NB):
                scatter_wait(b, NCH - NB + b)

        plsc.subcore_barrier()

        if D == 128:
            @pl.when(c == 0)
            def _():
                pltpu.sync_copy(acc_sh.at[pl.ds(s * RPT, RPT)],
                                out0.at[pl.ds(s * RPT, RPT)])

                @pl.when(s == NS - 1)
                def _():
                    pltpu.sync_copy(acc_sh.at[pl.ds(NS * RPT, RTAIL)],
                                    out0.at[pl.ds(NS * RPT, RTAIL)])

            @pl.when(c == 1)
            def _():
                pltpu.sync_copy(acc_sh.at[pl.ds(s * RPT, RPT)],
                                out1.at[pl.ds(s * RPT, RPT)])

                @pl.when(s == NS - 1)
                def _():
                    pltpu.sync_copy(acc_sh.at[pl.ds(NS * RPT, RTAIL)],
                                    out1.at[pl.ds(NS * RPT, RTAIL)])
        else:
            @pl.when(c == 0)
            def _():
                pltpu.sync_copy(acc_sh.at[pl.ds(s * RPT, RPT)],
                                out0.at[pl.ds(s * RPT, RPT), pl.ds(0, D)])

                @pl.when(s == NS - 1)
                def _():
                    pltpu.sync_copy(
                        acc_sh.at[pl.ds(NS * RPT, RTAIL)],
                        out0.at[pl.ds(NS * RPT, RTAIL), pl.ds(0, D)])

            @pl.when(c == 1)
            def _():
                pltpu.sync_copy(acc_sh.at[pl.ds(s * RPT, RPT)],
                                out0.at[pl.ds(s * RPT, RPT), pl.ds(D, D)])

                @pl.when(s == NS - 1)
                def _():
                    pltpu.sync_copy(
                        acc_sh.at[pl.ds(NS * RPT, RTAIL)],
                        out0.at[pl.ds(NS * RPT, RTAIL), pl.ds(D, D)])

    return k


_agg128 = _sc_edge_aggregate(128, True, K128, NB128)
_agg16 = _sc_edge_aggregate(16, True, K16, NB16)
_hist16 = _sc_edge_aggregate(16, False, K16, NB16)


_RB = 2000


def _tc_stage1(cnt, x, W1):
    def body(c_ref, x_ref, w_ref, hs_ref, dinv_ref):
        deg = c_ref[:, 0:1] + c_ref[:, 16:17] + 1.0
        dinv = lax.rsqrt(deg)
        h = jnp.dot(x_ref[...], w_ref[...], preferred_element_type=jnp.float32)
        hs_ref[...] = h * dinv
        dinv_ref[...] = dinv

    grid = (N // _RB,)
    return pl.pallas_call(
        body,
        grid=grid,
        in_specs=[
            pl.BlockSpec((_RB, 128), lambda i: (i, 0)),
            pl.BlockSpec((_RB, 128), lambda i: (i, 0)),
            pl.BlockSpec((128, 128), lambda i: (0, 0)),
        ],
        out_specs=[
            pl.BlockSpec((_RB, 128), lambda i: (i, 0)),
            pl.BlockSpec((_RB, 1), lambda i: (i, 0)),
        ],
        out_shape=[
            jax.ShapeDtypeStruct((N, 128), jnp.float32),
            jax.ShapeDtypeStruct((N, 1), jnp.float32),
        ],
    )(cnt, x, W1)


def _tc_stage2(p0, p1, hs1, dinv, b1, W2, Wc):
    def body(p0_ref, p1_ref, hs_ref, dinv_ref, b1_ref, w2_ref, wc_ref, out_ref):
        dinv = dinv_ref[...]
        y = (p0_ref[...] + p1_ref[...] + hs_ref[...]) * dinv + b1_ref[...]
        y = jnp.maximum(y, 0.0)
        h2 = jnp.dot(
            jnp.dot(y, w2_ref[...], preferred_element_type=jnp.float32),
            wc_ref[...], preferred_element_type=jnp.float32)
        hs2 = h2 * dinv
        out_ref[...] = jnp.pad(hs2, ((0, 0), (0, 14)))

    grid = (N // _RB,)
    return pl.pallas_call(
        body,
        grid=grid,
        in_specs=[
            pl.BlockSpec((_RB, 128), lambda i: (i, 0)),
            pl.BlockSpec((_RB, 128), lambda i: (i, 0)),
            pl.BlockSpec((_RB, 128), lambda i: (i, 0)),
            pl.BlockSpec((_RB, 1), lambda i: (i, 0)),
            pl.BlockSpec((1, 128), lambda i: (0, 0)),
            pl.BlockSpec((128, 64), lambda i: (0, 0)),
            pl.BlockSpec((64, 2), lambda i: (0, 0)),
        ],
        out_specs=pl.BlockSpec((_RB, 16), lambda i: (i, 0)),
        out_shape=jax.ShapeDtypeStruct((N, 16), jnp.float32),
    )(p0, p1, hs1, dinv, b1, W2, Wc)


def _tc_stage3(q, hs2p, dinv, b2, Wc, bc):
    def body(q_ref, hs_ref, dinv_ref, b2_ref, wc_ref, bc_ref, out_ref):
        agg = ((q_ref[:, 0:8] + q_ref[:, 16:24] + hs_ref[:, 0:8])
               * dinv_ref[...])
        b2c = jnp.dot(b2_ref[...], wc_ref[...],
                      preferred_element_type=jnp.float32) + bc_ref[...]
        out_ref[...] = agg[:, 0:2] + b2c

    grid = (N // _RB,)
    return pl.pallas_call(
        body,
        grid=grid,
        in_specs=[
            pl.BlockSpec((_RB, 128), lambda i: (i, 0)),
            pl.BlockSpec((_RB, 16), lambda i: (i, 0)),
            pl.BlockSpec((_RB, 1), lambda i: (i, 0)),
            pl.BlockSpec((1, 64), lambda i: (0, 0)),
            pl.BlockSpec((64, 2), lambda i: (0, 0)),
            pl.BlockSpec((1, 2), lambda i: (0, 0)),
        ],
        out_specs=pl.BlockSpec((_RB, 2), lambda i: (i, 0)),
        out_shape=jax.ShapeDtypeStruct((N, 2), jnp.float32),
    )(q, hs2p, dinv, b2, Wc, bc)


def _unwrap(res):
    return res[0] if isinstance(res, (list, tuple)) else res


def kernel(x, edge_index, W1, b1, W2, b2, Wc, bc):
    ei = edge_index.astype(jnp.int32)
    edge1 = ei.reshape(2, NW, EPT // K128, K128)
    edge2 = ei.reshape(2, NW, EPT // K16, K16)

    cnt = _unwrap(_hist16(edge2, x))

    hs1, dinv = _tc_stage1(cnt, x, W1)
    p0, p1 = _agg128(edge1, hs1)
    hs2p = _tc_stage2(p0, p1, hs1, dinv, b1.reshape(1, 128), W2, Wc)
    q = _unwrap(_agg16(edge2, hs2p))
    out = _tc_stage3(q, hs2p, dinv, b2.reshape(1, 64), Wc,
                     bc.reshape(1, 2))
    return out


# speedup vs baseline: 1.1141x; 1.0052x over previous
import functools

import jax
import jax.numpy as jnp
from jax import lax
from jax.experimental import pallas as pl
from jax.experimental.pallas import tpu as pltpu
from jax.experimental.pallas import tpu_sc as plsc

N = 10000
E = 320000
NC, NS = 2, 16
NW = NC * NS
EPT = E // NW
K128 = 40
NB128 = 5
K16 = 100
NB16 = 10
RPT = 624
RTAIL = N - NS * RPT

_MESH = plsc.VectorSubcoreMesh(
    core_axis_name="c", subcore_axis_name="s", num_cores=NC, num_subcores=NS
)


def _sc_edge_aggregate(D, with_gather, K, NB):
    NCH = EPT // K
    NOUTER = NCH // NB
    assert NCH * K == EPT and NOUTER * NB == NCH
    ZCOPIES = RPT // K
    ZREM = RPT % K
    scratch = [
        pltpu.VMEM((NCH, K), jnp.int32),
        pltpu.VMEM((NCH, K), jnp.int32),
        [pltpu.VMEM((K, D), jnp.float32) for _ in range(NB)],
        pltpu.VMEM_SHARED((N, D), jnp.float32),
        [pltpu.SemaphoreType.DMA for _ in range(NB)],
        [pltpu.SemaphoreType.DMA for _ in range(NB)],
    ]
    if D == 128:
        out_type = [
            jax.ShapeDtypeStruct((N, 128), jnp.float32),
            jax.ShapeDtypeStruct((N, 128), jnp.float32),
        ]
    else:
        out_type = [jax.ShapeDtypeStruct((N, 128), jnp.float32)]

    @functools.partial(
        pl.kernel, out_type=out_type, mesh=_MESH, scratch_types=scratch,
        compiler_params=pltpu.CompilerParams(use_tc_tiling_on_sc=False),
    )
    def k(edge_hbm, table_hbm, *rest):
        if D == 128:
            out0, out1 = rest[0], rest[1]
            src_all, dst_all, rows, acc_sh, sem_g, sem_s = rest[2:]
        else:
            out0 = rest[0]
            src_all, dst_all, rows, acc_sh, sem_g, sem_s = rest[1:]
        c = lax.axis_index("c")
        s = lax.axis_index("s")
        wid = s * NC + c

        zeros = jnp.zeros((16,), jnp.float32)

        def zrow(i, carry):
            for j in range(D // 16):
                rows[0][i, pl.ds(j * 16, 16)] = zeros
            return carry

        lax.fori_loop(0, K, zrow, 0)

        def zinit(j, n):
            return pltpu.make_async_copy(
                rows[0].at[pl.ds(0, n)],
                acc_sh.at[pl.ds(s * RPT + j * K, n)], sem_s[j % NB])

        for j in range(ZCOPIES):
            zinit(j, K).start()
        pltpu.sync_copy(rows[0].at[pl.ds(0, ZREM)],
                        acc_sh.at[pl.ds(s * RPT + ZCOPIES * K, ZREM)])

        @pl.when(s == NS - 1)
        def _():
            pltpu.sync_copy(rows[0].at[pl.ds(0, RTAIL)],
                            acc_sh.at[pl.ds(NS * RPT, RTAIL)])

        pltpu.sync_copy(edge_hbm.at[1, wid], dst_all)
        if with_gather:
            pltpu.sync_copy(edge_hbm.at[0, wid], src_all)
        for j in range(ZCOPIES):
            zinit(j, K).wait()

        if not with_gather:
            ones = jnp.ones((16,), jnp.float32)

            def orow(i, carry):
                for b in range(NB):
                    for j in range(D // 16):
                        rows[b][i, pl.ds(j * 16, 16)] = ones
                return carry

            lax.fori_loop(0, K, orow, 0)

        plsc.subcore_barrier()

        def gather_start(b, g):
            pltpu.async_copy(table_hbm.at[src_all.at[g]], rows[b], sem_g[b])

        def gather_wait(b, g):
            pltpu.make_async_copy(
                table_hbm.at[src_all.at[g]], rows[b], sem_g[b]).wait()

        def scatter_start(b, g):
            pltpu.async_copy(rows[b], acc_sh.at[dst_all.at[g]], sem_s[b],
                             add=True)

        def scatter_wait(b, g):
            pltpu.make_async_copy(
                rows[b], acc_sh.at[dst_all.at[g]], sem_s[b]).wait()

        if with_gather:
            for b in range(NB):
                gather_start(b, b)

            def body(t, carry):
                for b in range(NB):
                    g = t * NB + b
                    gather_wait(b, g)
                    scatter_start(b, g)

                    @pl.when(g + NB < NCH)
                    def _():
                        scatter_wait(b, g)
                        gather_start(b, g + NB)
                return carry

            lax.fori_loop(0, NOUTER, body, 0)
            for b in range(NB):
                scatter_wait(b, NCH - NB + b)
        else:
            def body(t, carry):
                for b in range(NB):
                    g = t * NB + b

                    @pl.when(t > 0)
                    def _():
                        scatter_wait(b, g)
                    scatter_start(b, g)
                return carry

            lax.fori_loop(0, NOUTER, body, 0)
            for b in range(NB):
                scatter_wait(b, NCH - NB + b)

        plsc.subcore_barrier()

        if D == 128:
            @pl.when(c == 0)
            def _():
                pltpu.sync_copy(acc_sh.at[pl.ds(s * RPT, RPT)],
                                out0.at[pl.ds(s * RPT, RPT)])

                @pl.when(s == NS - 1)
                def _():
                    pltpu.sync_copy(acc_sh.at[pl.ds(NS * RPT, RTAIL)],
                                    out0.at[pl.ds(NS * RPT, RTAIL)])

            @pl.when(c == 1)
            def _():
                pltpu.sync_copy(acc_sh.at[pl.ds(s * RPT, RPT)],
                                out1.at[pl.ds(s * RPT, RPT)])

                @pl.when(s == NS - 1)
                def _():
                    pltpu.sync_copy(acc_sh.at[pl.ds(NS * RPT, RTAIL)],
                                    out1.at[pl.ds(NS * RPT, RTAIL)])
        else:
            @pl.when(c == 0)
            def _():
                pltpu.sync_copy(acc_sh.at[pl.ds(s * RPT, RPT)],
                                out0.at[pl.ds(s * RPT, RPT), pl.ds(0, D)])

                @pl.when(s == NS - 1)
                def _():
                    pltpu.sync_copy(
                        acc_sh.at[pl.ds(NS * RPT, RTAIL)],
                        out0.at[pl.ds(NS * RPT, RTAIL), pl.ds(0, D)])

            @pl.when(c == 1)
            def _():
                pltpu.sync_copy(acc_sh.at[pl.ds(s * RPT, RPT)],
                                out0.at[pl.ds(s * RPT, RPT), pl.ds(D, D)])

                @pl.when(s == NS - 1)
                def _():
                    pltpu.sync_copy(
                        acc_sh.at[pl.ds(NS * RPT, RTAIL)],
                        out0.at[pl.ds(NS * RPT, RTAIL), pl.ds(D, D)])

    return k


_agg128 = _sc_edge_aggregate(128, True, K128, NB128)
_agg16 = _sc_edge_aggregate(16, True, K16, NB16)
_hist16 = _sc_edge_aggregate(16, False, K16, NB16)


_RB = 2000


def _tc_stage1(cnt, x, W1):
    def body(c_ref, x_ref, w_ref, hs_ref, dinv_ref):
        deg = c_ref[:, 0:1] + c_ref[:, 16:17] + 1.0
        dinv = lax.rsqrt(deg)
        h = jnp.dot(x_ref[...], w_ref[...], preferred_element_type=jnp.float32)
        hs_ref[...] = h * dinv
        dinv_ref[...] = dinv

    grid = (N // _RB,)
    return pl.pallas_call(
        body,
        grid=grid,
        in_specs=[
            pl.BlockSpec((_RB, 128), lambda i: (i, 0)),
            pl.BlockSpec((_RB, 128), lambda i: (i, 0)),
            pl.BlockSpec((128, 128), lambda i: (0, 0)),
        ],
        out_specs=[
            pl.BlockSpec((_RB, 128), lambda i: (i, 0)),
            pl.BlockSpec((_RB, 1), lambda i: (i, 0)),
        ],
        out_shape=[
            jax.ShapeDtypeStruct((N, 128), jnp.float32),
            jax.ShapeDtypeStruct((N, 1), jnp.float32),
        ],
    )(cnt, x, W1)


def _tc_stage2(p0, p1, hs1, dinv, b1, W2, Wc):
    def body(p0_ref, p1_ref, hs_ref, dinv_ref, b1_ref, w2_ref, wc_ref, out_ref):
        dinv = dinv_ref[...]
        y = (p0_ref[...] + p1_ref[...] + hs_ref[...]) * dinv + b1_ref[...]
        y = jnp.maximum(y, 0.0)
        h2 = jnp.dot(
            jnp.dot(y, w2_ref[...], preferred_element_type=jnp.float32),
            wc_ref[...], preferred_element_type=jnp.float32)
        hs2 = h2 * dinv
        out_ref[...] = jnp.pad(hs2, ((0, 0), (0, 14)))

    grid = (N // _RB,)
    return pl.pallas_call(
        body,
        grid=grid,
        in_specs=[
            pl.BlockSpec((_RB, 128), lambda i: (i, 0)),
            pl.BlockSpec((_RB, 128), lambda i: (i, 0)),
            pl.BlockSpec((_RB, 128), lambda i: (i, 0)),
            pl.BlockSpec((_RB, 1), lambda i: (i, 0)),
            pl.BlockSpec((1, 128), lambda i: (0, 0)),
            pl.BlockSpec((128, 64), lambda i: (0, 0)),
            pl.BlockSpec((64, 2), lambda i: (0, 0)),
        ],
        out_specs=pl.BlockSpec((_RB, 16), lambda i: (i, 0)),
        out_shape=jax.ShapeDtypeStruct((N, 16), jnp.float32),
    )(p0, p1, hs1, dinv, b1, W2, Wc)


def _tc_stage3(q, hs2p, dinv, b2, Wc, bc):
    def body(q_ref, hs_ref, dinv_ref, b2_ref, wc_ref, bc_ref, out_ref):
        agg = ((q_ref[:, 0:8] + q_ref[:, 16:24] + hs_ref[:, 0:8])
               * dinv_ref[...])
        b2c = jnp.dot(b2_ref[...], wc_ref[...],
                      preferred_element_type=jnp.float32) + bc_ref[...]
        out_ref[...] = agg[:, 0:2] + b2c

    grid = (N // _RB,)
    return pl.pallas_call(
        body,
        grid=grid,
        in_specs=[
            pl.BlockSpec((_RB, 128), lambda i: (i, 0)),
            pl.BlockSpec((_RB, 16), lambda i: (i, 0)),
            pl.BlockSpec((_RB, 1), lambda i: (i, 0)),
            pl.BlockSpec((1, 64), lambda i: (0, 0)),
            pl.BlockSpec((64, 2), lambda i: (0, 0)),
            pl.BlockSpec((1, 2), lambda i: (0, 0)),
        ],
        out_specs=pl.BlockSpec((_RB, 2), lambda i: (i, 0)),
        out_shape=jax.ShapeDtypeStruct((N, 2), jnp.float32),
    )(q, hs2p, dinv, b2, Wc, bc)


def _unwrap(res):
    return res[0] if isinstance(res, (list, tuple)) else res


def kernel(x, edge_index, W1, b1, W2, b2, Wc, bc):
    ei = edge_index.astype(jnp.int32)
    edge1 = ei.reshape(2, NW, EPT // K128, K128)
    edge2 = ei.reshape(2, NW, EPT // K16, K16)

    cnt = _unwrap(_hist16(edge2, x))

    hs1, dinv = _tc_stage1(cnt, x, W1)
    p0, p1 = _agg128(edge1, hs1)
    hs2p = _tc_stage2(p0, p1, hs1, dinv, b1.reshape(1, 128), W2, Wc)
    q = _unwrap(_agg16(edge2, hs2p))
    out = _tc_stage3(q, hs2p, dinv, b2.reshape(1, 64), Wc,
                     bc.reshape(1, 2))
    return out
